# baseline (speedup 1.0000x reference)
import functools

import jax
import jax.numpy as jnp
from jax import lax
from jax.experimental import pallas as pl
import jax.experimental.pallas.tpu as pltpu
from jax.experimental.pallas import tpu_sc as plsc

N = 10000
E = 320000
D = 128
G = 64

NC = 2
NS = 16
NW = NC * NS
EPW = E // NW
B = 80
CH = EPW // B
NBLK = 25
BLK = CH // NBLK
BLKE = BLK * B
RPT = 632
RPT_LAST = N - (NS - 1) * RPT
NG = B // 16

_mesh = plsc.VectorSubcoreMesh(core_axis_name="c", subcore_axis_name="s")
_sc_params = pltpu.CompilerParams(needs_layout_passes=False)
_f32 = jnp.float32


def _deg_body(dst_hbm, ew_hbm, z1_hbm, cnt_out, ews_out,
              idx_d2, ew_flat, ones_v, s_cnt, s_ews):
  cc = lax.axis_index("c")
  ss = lax.axis_index("s")
  wid = ss * NC + cc

  @pl.when(ss == 0)
  def _():
    pltpu.sync_copy(z1_hbm, s_cnt)
    pltpu.sync_copy(z1_hbm, s_ews)

  pltpu.sync_copy(dst_hbm.at[wid], idx_d2)
  pltpu.sync_copy(ew_hbm.at[wid], ew_flat)
  for i in range(B // 16):
    ones_v[pl.ds(i * 16, 16)] = jnp.ones((16,), _f32)
  plsc.subcore_barrier()

  def chunk(j, carry):
    pltpu.sync_copy(ones_v, s_cnt.at[idx_d2.at[j]], add=True)
    pltpu.sync_copy(ew_flat.at[pl.ds(j * B, B)], s_ews.at[idx_d2.at[j]],
                    add=True)
    return carry

  lax.fori_loop(0, CH, chunk, 0)
  plsc.subcore_barrier()

  @pl.when(ss == 0)
  def _():
    pltpu.sync_copy(s_cnt, cnt_out.at[cc])
    pltpu.sync_copy(s_ews, ews_out.at[cc])


_sc_degree = pl.kernel(
    _deg_body,
    out_type=[jax.ShapeDtypeStruct((NC, N), _f32),
              jax.ShapeDtypeStruct((NC, N), _f32)],
    mesh=_mesh,
    compiler_params=_sc_params,
    scratch_types=[
        pltpu.VMEM((CH, B), jnp.int32),
        pltpu.VMEM((EPW,), _f32),
        pltpu.VMEM((B,), _f32),
        pltpu.VMEM_SHARED((N,), _f32),
        pltpu.VMEM_SHARED((N,), _f32),
    ],
)


def _edge_body(xl_hbm, xr_hbm, m_hbm, src_hbm, dst_hbm, ew_hbm, z2_hbm, z1_hbm,
               att_hbm, we_hbm, num_out, den_out,
               idx_s2, idx_d2, ew_flat, att_v, we_v,
               xl_rows, xr_rows, scat_buf, p_v, m_vals,
               s_num, s_den, s_m,
               sem_gl, sem_gr, sem_gm, sem_sn, sem_sd):
  cc = lax.axis_index("c")
  ss = lax.axis_index("s")
  wid = ss * NC + cc

  @pl.when(ss < NS - 1)
  def _():
    pltpu.sync_copy(z2_hbm.at[pl.ds(ss * RPT, RPT)],
                    s_num.at[pl.ds(ss * RPT, RPT)])

  @pl.when(ss == NS - 1)
  def _():
    pltpu.sync_copy(z2_hbm.at[pl.ds((NS - 1) * RPT, RPT_LAST)],
                    s_num.at[pl.ds((NS - 1) * RPT, RPT_LAST)])

  @pl.when(ss == 0)
  def _():
    pltpu.sync_copy(z1_hbm, s_den)
    pltpu.sync_copy(m_hbm, s_m)

  pltpu.sync_copy(att_hbm, att_v)
  pltpu.sync_copy(we_hbm, we_v)
  plsc.subcore_barrier()

  iota16 = lax.iota(jnp.int32, 16)

  def drain_scatters():
    pltpu.make_async_copy(scat_buf, s_num.at[idx_d2.at[0]], sem_sn).wait()
    pltpu.make_async_copy(p_v, s_den.at[idx_d2.at[0]], sem_sd).wait()

  def block(bi, carry0):
    @pl.when(bi > 0)
    def _():
      drain_scatters()

    pltpu.sync_copy(src_hbm.at[wid, bi], idx_s2)
    pltpu.sync_copy(dst_hbm.at[wid, bi], idx_d2)
    pltpu.sync_copy(ew_hbm.at[wid, bi], ew_flat)

    def chunk(j, carry):
      gl = pltpu.async_copy(xl_hbm.at[idx_s2.at[j]], xl_rows, sem_gl)
      gr = pltpu.async_copy(xr_hbm.at[idx_d2.at[j]], xr_rows, sem_gr)
      gm = pltpu.async_copy(s_m.at[idx_d2.at[j]], m_vals, sem_gm)
      gl.wait()
      gr.wait()
      gm.wait()

      eg = [iota16 + (g * 16) for g in range(NG)]
      ew16 = [ew_flat[pl.ds(j * B + g * 16, 16)] for g in range(NG)]

      def dim_a(k, accs):
        k16 = jnp.full((16,), k, jnp.int32)
        attk = plsc.load_gather(att_v, [k16])
        wek = plsc.load_gather(we_v, [k16])
        out = []
        for g in range(NG):
          xlc = plsc.load_gather(xl_rows, [eg[g], k16])
          xrc = plsc.load_gather(xr_rows, [eg[g], k16])
          v = xlc + xrc + ew16[g] * wek
          v = jnp.maximum(v, 0.2 * v)
          out.append(accs[g] + attk * v)
        return tuple(out)

      accs = lax.fori_loop(0, D, dim_a,
                           tuple(jnp.zeros((16,), _f32) for _ in range(NG)))

      @pl.when(j > 0)
      def _():
        pltpu.make_async_copy(p_v, s_den.at[idx_d2.at[j]], sem_sd).wait()

      p16s = []
      for g in range(NG):
        m16 = m_vals[pl.ds(g * 16, 16)]
        p16 = jnp.exp(accs[g] - m16)
        p_v[pl.ds(g * 16, 16)] = p16
        p16s.append(p16)

      @pl.when(j > 0)
      def _():
        pltpu.make_async_copy(scat_buf, s_num.at[idx_d2.at[j]], sem_sn).wait()

      def dim_c(k, cy):
        k16 = jnp.full((16,), k, jnp.int32)
        for g in range(NG):
          xlc = plsc.load_gather(xl_rows, [eg[g], k16])
          plsc.store_scatter(scat_buf, [eg[g], k16], p16s[g] * xlc)
        return cy

      lax.fori_loop(0, D, dim_c, 0)

      pltpu.async_copy(scat_buf, s_num.at[idx_d2.at[j]], sem_sn, add=True)
      pltpu.async_copy(p_v, s_den.at[idx_d2.at[j]], sem_sd, add=True)
      return carry

    lax.fori_loop(0, BLK, chunk, 0)
    return carry0

  lax.fori_loop(0, NBLK, block, 0)
  drain_scatters()
  plsc.subcore_barrier()

  @pl.when(ss < NS - 1)
  def _():
    pltpu.sync_copy(s_num.at[pl.ds(ss * RPT, RPT)],
                    num_out.at[cc, pl.ds(ss * RPT, RPT)])

  @pl.when(ss == NS - 1)
  def _():
    pltpu.sync_copy(s_num.at[pl.ds((NS - 1) * RPT, RPT_LAST)],
                    num_out.at[cc, pl.ds((NS - 1) * RPT, RPT_LAST)])

  @pl.when(ss == 0)
  def _():
    pltpu.sync_copy(s_den, den_out.at[cc])


_sc_edge = pl.kernel(
    _edge_body,
    out_type=[jax.ShapeDtypeStruct((NC, N, D), _f32),
              jax.ShapeDtypeStruct((NC, N), _f32)],
    mesh=_mesh,
    compiler_params=_sc_params,
    scratch_types=[
        pltpu.VMEM((BLK, B), jnp.int32),
        pltpu.VMEM((BLK, B), jnp.int32),
        pltpu.VMEM((BLKE,), _f32),
        pltpu.VMEM((D,), _f32),
        pltpu.VMEM((D,), _f32),
        pltpu.VMEM((B, D), _f32),
        pltpu.VMEM((B, D), _f32),
        pltpu.VMEM((B, D), _f32),
        pltpu.VMEM((B,), _f32),
        pltpu.VMEM((B,), _f32),
        pltpu.VMEM_SHARED((N, D), _f32),
        pltpu.VMEM_SHARED((N,), _f32),
        pltpu.VMEM_SHARED((N,), _f32),
        pltpu.SemaphoreType.DMA,
        pltpu.SemaphoreType.DMA,
        pltpu.SemaphoreType.DMA,
        pltpu.SemaphoreType.DMA,
        pltpu.SemaphoreType.DMA,
    ],
)


def _loop_attr(cntT, ewsT):
  cnt = cntT[:, 0:1] + cntT[:, 1:2]
  ews = ewsT[:, 0:1] + ewsT[:, 1:2]
  return ews / jnp.maximum(cnt, 1.0)


def _self_score(xl, xr, la, we, att):
  z = xl + xr + la * we
  z = jnp.maximum(z, 0.2 * z)
  return jnp.sum(att * z, axis=1, keepdims=True)


def _prep_body(x_ref, wl_ref, bl_ref, wr_ref, br_ref, we_ref, att_ref,
               cntT_ref, ewsT_ref, xl_ref, xr_ref, m_ref):
  xv = x_ref[...]
  xl = jnp.dot(xv, wl_ref[...], preferred_element_type=_f32) + bl_ref[...]
  xr = jnp.dot(xv, wr_ref[...], preferred_element_type=_f32) + br_ref[...]
  la = _loop_attr(cntT_ref[...], ewsT_ref[...])
  xl_ref[...] = xl
  xr_ref[...] = xr
  m_ref[...] = _self_score(xl, xr, la, we_ref[...], att_ref[...])


_tc_prep = pl.pallas_call(
    _prep_body,
    out_shape=[jax.ShapeDtypeStruct((N, D), _f32),
               jax.ShapeDtypeStruct((N, D), _f32),
               jax.ShapeDtypeStruct((N, 1), _f32)],
)


def _merge(num_ref, denT_ref, xlp, bias):
  den = denT_ref[:, 0:1] + denT_ref[:, 1:2] + 1.0
  h = (num_ref[0] + num_ref[1] + xlp) / den + bias
  return jnp.maximum(h, 0.0)


def _mid_body(num_ref, denT_ref, xlp_ref, bias_ref, wl_ref, bl_ref, wr_ref,
              br_ref, we_ref, att_ref, cntT_ref, ewsT_ref,
              xl_ref, xr_ref, m_ref):
  h = _merge(num_ref, denT_ref, xlp_ref[...], bias_ref[...])
  xl = jnp.dot(h, wl_ref[...], preferred_element_type=_f32) + bl_ref[...]
  xr = jnp.dot(h, wr_ref[...], preferred_element_type=_f32) + br_ref[...]
  la = _loop_attr(cntT_ref[...], ewsT_ref[...])
  xl_ref[...] = xl
  xr_ref[...] = xr
  m_ref[...] = _self_score(xl, xr, la, we_ref[...], att_ref[...])


_tc_mid = pl.pallas_call(
    _mid_body,
    out_shape=[jax.ShapeDtypeStruct((N, D), _f32),
               jax.ShapeDtypeStruct((N, D), _f32),
               jax.ShapeDtypeStruct((N, 1), _f32)],
)


def _final_body(num_ref, denT_ref, xlp_ref, bias_ref, batch_ref, wlin_ref,
                blin_ref, out_ref):
  h = _merge(num_ref, denT_ref, xlp_ref[...], bias_ref[...])
  bb = batch_ref[...]
  oh = (lax.broadcasted_iota(jnp.int32, (G, N), 0) == bb).astype(_f32)
  cntb = jnp.sum(oh, axis=1, keepdims=True)
  pooled = jnp.dot(oh, h, preferred_element_type=_f32)
  pooled = pooled / jnp.maximum(cntb, 1.0)
  out_ref[...] = (jnp.dot(pooled, wlin_ref[...], preferred_element_type=_f32)
                  + blin_ref[...])


_tc_final = pl.pallas_call(
    _final_body,
    out_shape=jax.ShapeDtypeStruct((G, D), _f32),
)


def kernel(x, edge_index, edge_weight, batch, W1_l, b1_l, W1_r, b1_r, W1_e,
           att1, bias1, W2_l, b2_l, W2_r, b2_r, W2_e, att2, bias2,
           W_lin, b_lin):
  src5 = edge_index[0].reshape(NW, NBLK, BLK, B)
  dst5 = edge_index[1].reshape(NW, NBLK, BLK, B)
  dst3 = edge_index[1].reshape(NW, CH, B)
  ew2d = edge_weight.reshape(NW, EPW)
  ew3 = edge_weight.reshape(NW, NBLK, BLKE)
  z1 = jnp.zeros((N,), _f32)
  z2 = jnp.zeros((N, D), _f32)

  cnt_p, ews_p = _sc_degree(dst3, ew2d, z1)
  cntT, ewsT = cnt_p.T, ews_p.T

  xl1, xr1, m1 = _tc_prep(x, W1_l, b1_l.reshape(1, D), W1_r,
                          b1_r.reshape(1, D), W1_e.reshape(1, D),
                          att1.reshape(1, D), cntT, ewsT)
  num1, den1 = _sc_edge(xl1, xr1, m1.reshape(N), src5, dst5, ew3, z2, z1,
                        att1, W1_e.reshape(D))

  xl2, xr2, m2 = _tc_mid(num1, den1.T, xl1, bias1.reshape(1, D), W2_l,
                         b2_l.reshape(1, D), W2_r, b2_r.reshape(1, D),
                         W2_e.reshape(1, D), att2.reshape(1, D), cntT, ewsT)
  num2, den2 = _sc_edge(xl2, xr2, m2.reshape(N), src5, dst5, ew3, z2, z1,
                        att2, W2_e.reshape(D))

  return _tc_final(num2, den2.T, xl2, bias2.reshape(1, D),
                   batch.reshape(1, N), W_lin, b_lin)


# speedup vs baseline: 4.9609x; 4.9609x over previous
import functools

import jax
import jax.numpy as jnp
from jax import lax
from jax.experimental import pallas as pl
import jax.experimental.pallas.tpu as pltpu
from jax.experimental.pallas import tpu_sc as plsc

N = 10000
E = 320000
D = 128
G = 64

NC = 2
NS = 16
NW = NC * NS
EPW = E // NW
B = 80
CH = EPW // B
NBLK = 25
BLK = CH // NBLK
BLKE = BLK * B
RPT = 632
RPT_LAST = N - (NS - 1) * RPT
NG = B // 16

_mesh = plsc.VectorSubcoreMesh(core_axis_name="c", subcore_axis_name="s")
_sc_params = pltpu.CompilerParams(needs_layout_passes=False)
_f32 = jnp.float32


def _deg_body(dst_hbm, ew_hbm, z1_hbm, cnt_out, ews_out,
              idx_d2, ew_flat, ones_v, s_cnt, s_ews):
  cc = lax.axis_index("c")
  ss = lax.axis_index("s")
  wid = ss * NC + cc

  @pl.when(ss == 0)
  def _():
    pltpu.sync_copy(z1_hbm, s_cnt)
    pltpu.sync_copy(z1_hbm, s_ews)

  pltpu.sync_copy(dst_hbm.at[wid], idx_d2)
  pltpu.sync_copy(ew_hbm.at[wid], ew_flat)
  for i in range(B // 16):
    ones_v[pl.ds(i * 16, 16)] = jnp.ones((16,), _f32)
  plsc.subcore_barrier()

  def chunk(j, carry):
    pltpu.sync_copy(ones_v, s_cnt.at[idx_d2.at[j]], add=True)
    pltpu.sync_copy(ew_flat.at[pl.ds(j * B, B)], s_ews.at[idx_d2.at[j]],
                    add=True)
    return carry

  lax.fori_loop(0, CH, chunk, 0)
  plsc.subcore_barrier()

  @pl.when(ss == 0)
  def _():
    pltpu.sync_copy(s_cnt, cnt_out.at[cc])
    pltpu.sync_copy(s_ews, ews_out.at[cc])


_sc_degree = pl.kernel(
    _deg_body,
    out_type=[jax.ShapeDtypeStruct((NC, N), _f32),
              jax.ShapeDtypeStruct((NC, N), _f32)],
    mesh=_mesh,
    compiler_params=_sc_params,
    scratch_types=[
        pltpu.VMEM((CH, B), jnp.int32),
        pltpu.VMEM((EPW,), _f32),
        pltpu.VMEM((B,), _f32),
        pltpu.VMEM_SHARED((N,), _f32),
        pltpu.VMEM_SHARED((N,), _f32),
    ],
)


def _edge_body(xl_hbm, xr_hbm, m_hbm, src_hbm, dst_hbm, ew_hbm, z2_hbm, z1_hbm,
               att_hbm, we_hbm, num_out, den_out,
               idx_s2, idx_d2, ew_flat, att_v, we_v,
               xl_rows, xr_rows, scat_buf, t_flat, p_v, m_vals,
               s_num, s_den, s_m,
               sem_gl, sem_gr, sem_gm, sem_sn, sem_sd):
  cc = lax.axis_index("c")
  ss = lax.axis_index("s")
  wid = ss * NC + cc

  @pl.when(ss < NS - 1)
  def _():
    pltpu.sync_copy(z2_hbm.at[pl.ds(ss * RPT, RPT)],
                    s_num.at[pl.ds(ss * RPT, RPT)])

  @pl.when(ss == NS - 1)
  def _():
    pltpu.sync_copy(z2_hbm.at[pl.ds((NS - 1) * RPT, RPT_LAST)],
                    s_num.at[pl.ds((NS - 1) * RPT, RPT_LAST)])

  @pl.when(ss == 0)
  def _():
    pltpu.sync_copy(z1_hbm, s_den)
    pltpu.sync_copy(m_hbm, s_m)

  pltpu.sync_copy(att_hbm, att_v)
  pltpu.sync_copy(we_hbm, we_v)
  plsc.subcore_barrier()

  iota16 = lax.iota(jnp.int32, 16)
  attc = [att_v[pl.ds(k * 16, 16)] for k in range(8)]
  wec = [we_v[pl.ds(k * 16, 16)] for k in range(8)]

  def drain_scatters():
    pltpu.make_async_copy(scat_buf, s_num.at[idx_d2.at[0]], sem_sn).wait()
    pltpu.make_async_copy(p_v, s_den.at[idx_d2.at[0]], sem_sd).wait()

  def block(bi, carry0):
    @pl.when(bi > 0)
    def _():
      drain_scatters()

    pltpu.sync_copy(src_hbm.at[wid, bi], idx_s2)
    pltpu.sync_copy(dst_hbm.at[wid, bi], idx_d2)
    pltpu.sync_copy(ew_hbm.at[wid, bi], ew_flat)

    def chunk(j, carry):
      gl = pltpu.async_copy(xl_hbm.at[idx_s2.at[j]], xl_rows, sem_gl)
      gr = pltpu.async_copy(xr_hbm.at[idx_d2.at[j]], xr_rows, sem_gr)
      gm = pltpu.async_copy(s_m.at[idx_d2.at[j]], m_vals, sem_gm)
      gl.wait()
      gr.wait()

      def grp_a(g, cy):
        ewg = ew_flat[pl.ds(j * B + g * 16, 16)]
        for e2 in range(16):
          e = g * 16 + e2
          ew16 = jnp.full((16,), ewg[e2], _f32)
          pacc = [jnp.zeros((16,), _f32) for _ in range(4)]
          for k in range(8):
            v = (xl_rows[e, pl.ds(k * 16, 16)] + xr_rows[e, pl.ds(k * 16, 16)]
                 + ew16 * wec[k])
            v = jnp.maximum(v, 0.2 * v)
            pacc[k % 4] = pacc[k % 4] + attc[k] * v
          t_flat[pl.ds(e * 16, 16)] = (pacc[0] + pacc[1]) + (pacc[2] + pacc[3])
        return cy

      lax.fori_loop(0, NG, grp_a, 0)

      @pl.when(j > 0)
      def _():
        pltpu.make_async_copy(p_v, s_den.at[idx_d2.at[j]], sem_sd).wait()
      gm.wait()

      for g in range(B // 16):
        base16 = (iota16 + (g * 16)) * 16
        sc_acc = jnp.zeros((16,), _f32)
        for l in range(16):
          sc_acc = sc_acc + plsc.load_gather(t_flat, [base16 + l])
        m16 = m_vals[pl.ds(g * 16, 16)]
        p_v[pl.ds(g * 16, 16)] = jnp.exp(sc_acc - m16)

      @pl.when(j > 0)
      def _():
        pltpu.make_async_copy(scat_buf, s_num.at[idx_d2.at[j]], sem_sn).wait()

      def grp_c(g, cy):
        pg = p_v[pl.ds(g * 16, 16)]
        for e2 in range(16):
          e = g * 16 + e2
          p16 = jnp.full((16,), pg[e2], _f32)
          for k in range(8):
            scat_buf[e, pl.ds(k * 16, 16)] = (
                p16 * xl_rows[e, pl.ds(k * 16, 16)])
        return cy

      lax.fori_loop(0, NG, grp_c, 0)

      pltpu.async_copy(scat_buf, s_num.at[idx_d2.at[j]], sem_sn, add=True)
      pltpu.async_copy(p_v, s_den.at[idx_d2.at[j]], sem_sd, add=True)
      return carry

    lax.fori_loop(0, BLK, chunk, 0)
    return carry0

  lax.fori_loop(0, NBLK, block, 0)
  drain_scatters()
  plsc.subcore_barrier()

  @pl.when(ss < NS - 1)
  def _():
    pltpu.sync_copy(s_num.at[pl.ds(ss * RPT, RPT)],
                    num_out.at[cc, pl.ds(ss * RPT, RPT)])

  @pl.when(ss == NS - 1)
  def _():
    pltpu.sync_copy(s_num.at[pl.ds((NS - 1) * RPT, RPT_LAST)],
                    num_out.at[cc, pl.ds((NS - 1) * RPT, RPT_LAST)])

  @pl.when(ss == 0)
  def _():
    pltpu.sync_copy(s_den, den_out.at[cc])


_sc_edge = pl.kernel(
    _edge_body,
    out_type=[jax.ShapeDtypeStruct((NC, N, D), _f32),
              jax.ShapeDtypeStruct((NC, N), _f32)],
    mesh=_mesh,
    compiler_params=_sc_params,
    scratch_types=[
        pltpu.VMEM((BLK, B), jnp.int32),
        pltpu.VMEM((BLK, B), jnp.int32),
        pltpu.VMEM((BLKE,), _f32),
        pltpu.VMEM((D,), _f32),
        pltpu.VMEM((D,), _f32),
        pltpu.VMEM((B, D), _f32),
        pltpu.VMEM((B, D), _f32),
        pltpu.VMEM((B, D), _f32),
        pltpu.VMEM((B * 16,), _f32),
        pltpu.VMEM((B,), _f32),
        pltpu.VMEM((B,), _f32),
        pltpu.VMEM_SHARED((N, D), _f32),
        pltpu.VMEM_SHARED((N,), _f32),
        pltpu.VMEM_SHARED((N,), _f32),
        pltpu.SemaphoreType.DMA,
        pltpu.SemaphoreType.DMA,
        pltpu.SemaphoreType.DMA,
        pltpu.SemaphoreType.DMA,
        pltpu.SemaphoreType.DMA,
    ],
)


def _loop_attr(cntT, ewsT):
  cnt = cntT[:, 0:1] + cntT[:, 1:2]
  ews = ewsT[:, 0:1] + ewsT[:, 1:2]
  return ews / jnp.maximum(cnt, 1.0)


def _self_score(xl, xr, la, we, att):
  z = xl + xr + la * we
  z = jnp.maximum(z, 0.2 * z)
  return jnp.sum(att * z, axis=1, keepdims=True)


def _prep_body(x_ref, wl_ref, bl_ref, wr_ref, br_ref, we_ref, att_ref,
               cntT_ref, ewsT_ref, xl_ref, xr_ref, m_ref):
  xv = x_ref[...]
  xl = jnp.dot(xv, wl_ref[...], preferred_element_type=_f32) + bl_ref[...]
  xr = jnp.dot(xv, wr_ref[...], preferred_element_type=_f32) + br_ref[...]
  la = _loop_attr(cntT_ref[...], ewsT_ref[...])
  xl_ref[...] = xl
  xr_ref[...] = xr
  m_ref[...] = _self_score(xl, xr, la, we_ref[...], att_ref[...])


_tc_prep = pl.pallas_call(
    _prep_body,
    out_shape=[jax.ShapeDtypeStruct((N, D), _f32),
               jax.ShapeDtypeStruct((N, D), _f32),
               jax.ShapeDtypeStruct((N, 1), _f32)],
)


def _merge(num_ref, denT_ref, xlp, bias):
  den = denT_ref[:, 0:1] + denT_ref[:, 1:2] + 1.0
  h = (num_ref[0] + num_ref[1] + xlp) / den + bias
  return jnp.maximum(h, 0.0)


def _mid_body(num_ref, denT_ref, xlp_ref, bias_ref, wl_ref, bl_ref, wr_ref,
              br_ref, we_ref, att_ref, cntT_ref, ewsT_ref,
              xl_ref, xr_ref, m_ref):
  h = _merge(num_ref, denT_ref, xlp_ref[...], bias_ref[...])
  xl = jnp.dot(h, wl_ref[...], preferred_element_type=_f32) + bl_ref[...]
  xr = jnp.dot(h, wr_ref[...], preferred_element_type=_f32) + br_ref[...]
  la = _loop_attr(cntT_ref[...], ewsT_ref[...])
  xl_ref[...] = xl
  xr_ref[...] = xr
  m_ref[...] = _self_score(xl, xr, la, we_ref[...], att_ref[...])


_tc_mid = pl.pallas_call(
    _mid_body,
    out_shape=[jax.ShapeDtypeStruct((N, D), _f32),
               jax.ShapeDtypeStruct((N, D), _f32),
               jax.ShapeDtypeStruct((N, 1), _f32)],
)


def _final_body(num_ref, denT_ref, xlp_ref, bias_ref, batch_ref, wlin_ref,
                blin_ref, out_ref):
  h = _merge(num_ref, denT_ref, xlp_ref[...], bias_ref[...])
  bb = batch_ref[...]
  oh = (lax.broadcasted_iota(jnp.int32, (G, N), 0) == bb).astype(_f32)
  cntb = jnp.sum(oh, axis=1, keepdims=True)
  pooled = jnp.dot(oh, h, preferred_element_type=_f32)
  pooled = pooled / jnp.maximum(cntb, 1.0)
  out_ref[...] = (jnp.dot(pooled, wlin_ref[...], preferred_element_type=_f32)
                  + blin_ref[...])


_tc_final = pl.pallas_call(
    _final_body,
    out_shape=jax.ShapeDtypeStruct((G, D), _f32),
)


def kernel(x, edge_index, edge_weight, batch, W1_l, b1_l, W1_r, b1_r, W1_e,
           att1, bias1, W2_l, b2_l, W2_r, b2_r, W2_e, att2, bias2,
           W_lin, b_lin):
  src5 = edge_index[0].reshape(NW, NBLK, BLK, B)
  dst5 = edge_index[1].reshape(NW, NBLK, BLK, B)
  dst3 = edge_index[1].reshape(NW, CH, B)
  ew2d = edge_weight.reshape(NW, EPW)
  ew3 = edge_weight.reshape(NW, NBLK, BLKE)
  z1 = jnp.zeros((N,), _f32)
  z2 = jnp.zeros((N, D), _f32)

  cnt_p, ews_p = _sc_degree(dst3, ew2d, z1)
  cntT, ewsT = cnt_p.T, ews_p.T

  xl1, xr1, m1 = _tc_prep(x, W1_l, b1_l.reshape(1, D), W1_r,
                          b1_r.reshape(1, D), W1_e.reshape(1, D),
                          att1.reshape(1, D), cntT, ewsT)
  num1, den1 = _sc_edge(xl1, xr1, m1.reshape(N), src5, dst5, ew3, z2, z1,
                        att1, W1_e.reshape(D))

  xl2, xr2, m2 = _tc_mid(num1, den1.T, xl1, bias1.reshape(1, D), W2_l,
                         b2_l.reshape(1, D), W2_r, b2_r.reshape(1, D),
                         W2_e.reshape(1, D), att2.reshape(1, D), cntT, ewsT)
  num2, den2 = _sc_edge(xl2, xr2, m2.reshape(N), src5, dst5, ew3, z2, z1,
                        att2, W2_e.reshape(D))

  return _tc_final(num2, den2.T, xl2, bias2.reshape(1, D),
                   batch.reshape(1, N), W_lin, b_lin)
